# baseline (speedup 1.0000x reference)
import functools

import jax
import jax.numpy as jnp
from jax import lax
from jax.experimental import pallas as pl
from jax.experimental.pallas import tpu as pltpu
from jax.experimental.pallas import tpu_sc as plsc

B = 16384
NNEG = 20
V = 1000000
D = 64
L = 16

NW = 32
BPW = B // NW
C = 32
NCHUNK = BPW // C
GPC = C // L
NEGC = C * NNEG
IDXW = 40
NEGR = NEGC // IDXW
NEG_ROWS_TOT = B * NNEG // IDXW

VSTRIDE = D + 1
NSLAB = NNEG * D + 1

LN2 = 0.6931471805599453
SQRTH = 0.7071067811865476


def _vlog(x):
    bits = lax.bitcast_convert_type(x, jnp.int32)
    e = lax.shift_right_logical(bits, 23) - 127
    m = lax.bitcast_convert_type(
        (bits & 0x007FFFFF) | 0x3F800000, jnp.float32)
    ef = e.astype(jnp.float32)
    small = m < (2.0 * SQRTH)
    ef = jnp.where(small, ef, ef + 1.0)
    m = jnp.where(small, m, 0.5 * m)
    r = m - 1.0
    z = r * r
    p = 7.0376836292e-2
    p = p * r + -1.1514610310e-1
    p = p * r + 1.1676998740e-1
    p = p * r + -1.2420140846e-1
    p = p * r + 1.4249322787e-1
    p = p * r + -1.6668057665e-1
    p = p * r + 2.0000714765e-1
    p = p * r + -2.4999993993e-1
    p = p * r + 3.3333331174e-1
    y = r * z * p - 0.5 * z + r
    return y + ef * LN2


def _make_sc_kernel():
    mesh = plsc.VectorSubcoreMesh(core_axis_name="c", subcore_axis_name="s")

    @functools.partial(
        pl.kernel,
        out_type=(
            jax.ShapeDtypeStruct((B,), jnp.float32),
            jax.ShapeDtypeStruct((B,), jnp.float32),
        ),
        mesh=mesh,
        compiler_params=pltpu.CompilerParams(
            use_tc_tiling_on_sc=False, needs_layout_passes=False),
        scratch_types=[
            pltpu.VMEM((C,), jnp.int32),
            pltpu.VMEM((C,), jnp.int32),
            pltpu.VMEM((C,), jnp.float32),
            pltpu.VMEM((NEGR, IDXW), jnp.int32),
            pltpu.VMEM((C, D), jnp.float32),
            pltpu.VMEM((C, D), jnp.float32),
            pltpu.VMEM((C, D), jnp.float32),
            pltpu.VMEM((NEGC, D), jnp.float32),
            pltpu.VMEM((C * VSTRIDE,), jnp.float32),
            pltpu.VMEM((C * VSTRIDE,), jnp.float32),
            pltpu.VMEM((C * VSTRIDE,), jnp.float32),
            pltpu.VMEM((C * NSLAB,), jnp.float32),
            pltpu.VMEM((C,), jnp.float32),
            pltpu.VMEM((C,), jnp.float32),
            pltpu.SemaphoreType.DMA,
        ],
    )
    def sc_kernel(pos_v_hbm, pos_u_hbm, w_hbm, neg_hbm, emb_hbm, ctx_hbm,
                  out1_hbm, out2_hbm,
                  idxv, idxu, wbuf, negidx, vraw, u1raw, u2raw, negraw,
                  vpad, u1pad, u2pad, negpad, o1, o2, sem):
        wid = lax.axis_index("s") * 2 + lax.axis_index("c")
        base = wid * BPW
        rbase = wid * (BPW * NNEG // IDXW)
        iota = lax.iota(jnp.int32, L)

        def stage_idx(ci):
            b0 = pl.multiple_of(base + ci * C, C)
            r0 = rbase + ci * NEGR
            pltpu.sync_copy(pos_v_hbm.at[pl.ds(b0, C)], idxv)
            pltpu.sync_copy(pos_u_hbm.at[pl.ds(b0, C)], idxu)
            pltpu.sync_copy(w_hbm.at[pl.ds(b0, C)], wbuf)
            pltpu.sync_copy(neg_hbm.at[pl.ds(r0, NEGR)], negidx)

        def issue(ci):
            stage_idx(ci)
            pltpu.async_copy(emb_hbm.at[idxv], vraw, sem)
            pltpu.async_copy(emb_hbm.at[idxu], u1raw, sem)
            pltpu.async_copy(ctx_hbm.at[idxu], u2raw, sem)
            for k in range(NEGR):
                pltpu.async_copy(ctx_hbm.at[negidx.at[k]],
                                 negraw.at[pl.ds(k * IDXW, IDXW)], sem)

        def drain():
            pltpu.make_async_copy(emb_hbm.at[idxv], vraw, sem).wait()
            pltpu.make_async_copy(emb_hbm.at[idxu], u1raw, sem).wait()
            pltpu.make_async_copy(ctx_hbm.at[idxu], u2raw, sem).wait()
            for k in range(NEGR):
                pltpu.make_async_copy(ctx_hbm.at[negidx.at[k]],
                                      negraw.at[pl.ds(k * IDXW, IDXW)],
                                      sem).wait()

        def relayout():
            @plsc.parallel_loop(0, C, 1, unroll=2)
            def rl(b):
                for k in range(D // L):
                    dst = iota + (b * VSTRIDE + k * L)
                    plsc.store_scatter(vpad, [dst], vraw.at[b][pl.ds(k * L, L)])
                    plsc.store_scatter(u1pad, [dst],
                                       u1raw.at[b][pl.ds(k * L, L)])
                    plsc.store_scatter(u2pad, [dst],
                                       u2raw.at[b][pl.ds(k * L, L)])
                nb = b * NSLAB
                for n in range(NNEG):
                    src = negraw.at[b * NNEG + n]
                    for k in range(D // L):
                        plsc.store_scatter(
                            negpad, [iota + (nb + (n * D + k * L))],
                            src[pl.ds(k * L, L)])

        def compute_store(ci, wgs):
            b0 = pl.multiple_of(base + ci * C, C)
            zero = jnp.zeros((L,), jnp.float32)
            for g in range(GPC):
                bi = iota + (g * L)
                vbase = bi * VSTRIDE
                nbase = bi * NSLAB
                NH = NNEG // 2

                def dbody_a(dd, acc):
                    acc1, acc2, ts = acc
                    v_d = plsc.load_gather(vpad, [vbase + dd])
                    u1_d = plsc.load_gather(u1pad, [vbase + dd])
                    u2_d = plsc.load_gather(u2pad, [vbase + dd])
                    acc1 = acc1 + v_d * u1_d
                    acc2 = acc2 + v_d * u2_d
                    ts = tuple(
                        ts[n] + v_d * plsc.load_gather(
                            negpad, [nbase + (dd + n * D)])
                        for n in range(NH))
                    return acc1, acc2, ts

                def dbody_b(dd, ts):
                    v_d = plsc.load_gather(vpad, [vbase + dd])
                    return tuple(
                        ts[n] + v_d * plsc.load_gather(
                            negpad, [nbase + (dd + (NH + n) * D)])
                        for n in range(NNEG - NH))

                s1, s2, ts_a = lax.fori_loop(
                    0, D, dbody_a, (zero, zero, (zero,) * NH))
                ts_b = lax.fori_loop(0, D, dbody_b, (zero,) * (NNEG - NH))
                ts = ts_a + ts_b

                smax = zero
                prod = jnp.full((L,), 1.0, jnp.float32)
                for t in ts:
                    smax = smax + jnp.maximum(t, 0.0)
                    prod = prod * (1.0 + jnp.exp(-jnp.abs(t)))
                f1 = 1.0 + jnp.exp(-jnp.abs(s1))
                f2 = 1.0 + jnp.exp(-jnp.abs(s2))
                wg = wgs[g]
                sc1 = (jnp.maximum(-s1, 0.0) + smax + _vlog(prod * f1)) * wg
                sc2 = (jnp.maximum(-s2, 0.0) + smax + _vlog(prod * f2)) * wg
                o1[pl.ds(g * L, L)] = sc1
                o2[pl.ds(g * L, L)] = sc2

            pltpu.sync_copy(o1, out1_hbm.at[pl.ds(b0, C)])
            pltpu.sync_copy(o2, out2_hbm.at[pl.ds(b0, C)])

        issue(0)

        def chunk_body(ci, carry):
            drain()
            relayout()
            wgs = [wbuf[pl.ds(g * L, L)] for g in range(GPC)]

            @pl.when(ci < NCHUNK - 1)
            def _issue_next():
                issue(ci + 1)

            compute_store(ci, wgs)
            return carry

        lax.fori_loop(0, NCHUNK, chunk_body, 0)

    return sc_kernel


_SC_KERNEL = _make_sc_kernel()


@jax.jit
def kernel(pos, neg, W_emb, W_ctx):
    pos_v = pos[:, 0].astype(jnp.int32)
    pos_u = pos[:, 1].astype(jnp.int32)
    w = pos[:, 2]
    neg2d = neg.astype(jnp.int32).reshape(NEG_ROWS_TOT, IDXW)
    return _SC_KERNEL(pos_v, pos_u, w, neg2d, W_emb, W_ctx)


# speedup vs baseline: 1.0170x; 1.0170x over previous
import functools

import jax
import jax.numpy as jnp
from jax import lax
from jax.experimental import pallas as pl
from jax.experimental.pallas import tpu as pltpu
from jax.experimental.pallas import tpu_sc as plsc

B = 16384
NNEG = 20
V = 1000000
D = 64
L = 16

NW = 32
BPW = B // NW
C = 32
NCHUNK = BPW // C
GPC = C // L
NEGC = C * NNEG
IDXW = 40
NEGR = NEGC // IDXW
NEG_ROWS_TOT = B * NNEG // IDXW

VSTRIDE = D + 1
NSLAB = NNEG * D + 1

LN2 = 0.6931471805599453
SQRTH = 0.7071067811865476


def _vlog(x):
    bits = lax.bitcast_convert_type(x, jnp.int32)
    e = lax.shift_right_logical(bits, 23) - 127
    m = lax.bitcast_convert_type(
        (bits & 0x007FFFFF) | 0x3F800000, jnp.float32)
    ef = e.astype(jnp.float32)
    small = m < (2.0 * SQRTH)
    ef = jnp.where(small, ef, ef + 1.0)
    m = jnp.where(small, m, 0.5 * m)
    r = m - 1.0
    z = r * r
    p = 7.0376836292e-2
    p = p * r + -1.1514610310e-1
    p = p * r + 1.1676998740e-1
    p = p * r + -1.2420140846e-1
    p = p * r + 1.4249322787e-1
    p = p * r + -1.6668057665e-1
    p = p * r + 2.0000714765e-1
    p = p * r + -2.4999993993e-1
    p = p * r + 3.3333331174e-1
    y = r * z * p - 0.5 * z + r
    return y + ef * LN2


def _make_sc_kernel():
    mesh = plsc.VectorSubcoreMesh(core_axis_name="c", subcore_axis_name="s")

    @functools.partial(
        pl.kernel,
        out_type=(
            jax.ShapeDtypeStruct((B,), jnp.float32),
            jax.ShapeDtypeStruct((B,), jnp.float32),
        ),
        mesh=mesh,
        compiler_params=pltpu.CompilerParams(
            use_tc_tiling_on_sc=False, needs_layout_passes=False),
        scratch_types=[
            pltpu.VMEM((C,), jnp.int32),
            pltpu.VMEM((C,), jnp.int32),
            pltpu.VMEM((C,), jnp.float32),
            pltpu.VMEM((NEGR, IDXW), jnp.int32),
            pltpu.VMEM((C, D), jnp.float32),
            pltpu.VMEM((C, D), jnp.float32),
            pltpu.VMEM((C, D), jnp.float32),
            pltpu.VMEM((NEGC, D), jnp.float32),
            pltpu.VMEM((C * VSTRIDE,), jnp.float32),
            pltpu.VMEM((C * VSTRIDE,), jnp.float32),
            pltpu.VMEM((C * VSTRIDE,), jnp.float32),
            pltpu.VMEM((C * NSLAB,), jnp.float32),
            pltpu.VMEM((C,), jnp.float32),
            pltpu.VMEM((C,), jnp.float32),
            pltpu.SemaphoreType.DMA,
        ],
    )
    def sc_kernel(pos_v_hbm, pos_u_hbm, w_hbm, neg_hbm, emb_hbm, ctx_hbm,
                  out1_hbm, out2_hbm,
                  idxv, idxu, wbuf, negidx, vraw, u1raw, u2raw, negraw,
                  vpad, u1pad, u2pad, negpad, o1, o2, sem):
        wid = lax.axis_index("s") * 2 + lax.axis_index("c")
        base = wid * BPW
        rbase = wid * (BPW * NNEG // IDXW)
        iota = lax.iota(jnp.int32, L)

        def stage_idx(ci):
            b0 = pl.multiple_of(base + ci * C, C)
            r0 = rbase + ci * NEGR
            cps = [
                pltpu.async_copy(pos_v_hbm.at[pl.ds(b0, C)], idxv, sem),
                pltpu.async_copy(pos_u_hbm.at[pl.ds(b0, C)], idxu, sem),
                pltpu.async_copy(w_hbm.at[pl.ds(b0, C)], wbuf, sem),
                pltpu.async_copy(neg_hbm.at[pl.ds(r0, NEGR)], negidx, sem),
            ]
            for cp in cps:
                cp.wait()

        def issue(ci):
            stage_idx(ci)
            pltpu.async_copy(emb_hbm.at[idxv], vraw, sem)
            pltpu.async_copy(emb_hbm.at[idxu], u1raw, sem)
            pltpu.async_copy(ctx_hbm.at[idxu], u2raw, sem)
            for k in range(NEGR):
                pltpu.async_copy(ctx_hbm.at[negidx.at[k]],
                                 negraw.at[pl.ds(k * IDXW, IDXW)], sem)

        def drain():
            pltpu.make_async_copy(emb_hbm.at[idxv], vraw, sem).wait()
            pltpu.make_async_copy(emb_hbm.at[idxu], u1raw, sem).wait()
            pltpu.make_async_copy(ctx_hbm.at[idxu], u2raw, sem).wait()
            for k in range(NEGR):
                pltpu.make_async_copy(ctx_hbm.at[negidx.at[k]],
                                      negraw.at[pl.ds(k * IDXW, IDXW)],
                                      sem).wait()

        def relayout():
            @plsc.parallel_loop(0, C, 1, unroll=2)
            def rl(b):
                for k in range(D // L):
                    dst = iota + (b * VSTRIDE + k * L)
                    plsc.store_scatter(vpad, [dst], vraw.at[b][pl.ds(k * L, L)])
                    plsc.store_scatter(u1pad, [dst],
                                       u1raw.at[b][pl.ds(k * L, L)])
                    plsc.store_scatter(u2pad, [dst],
                                       u2raw.at[b][pl.ds(k * L, L)])
                nb = b * NSLAB
                for n in range(NNEG):
                    src = negraw.at[b * NNEG + n]
                    for k in range(D // L):
                        plsc.store_scatter(
                            negpad, [iota + (nb + (n * D + k * L))],
                            src[pl.ds(k * L, L)])

        def compute_store(ci, wgs):
            b0 = pl.multiple_of(base + ci * C, C)
            zero = jnp.zeros((L,), jnp.float32)
            for g in range(GPC):
                bi = iota + (g * L)
                vbase = bi * VSTRIDE
                nbase = bi * NSLAB
                NH = NNEG // 2

                def dbody_a(dd, acc):
                    acc1, acc2, ts = acc
                    v_d = plsc.load_gather(vpad, [vbase + dd])
                    u1_d = plsc.load_gather(u1pad, [vbase + dd])
                    u2_d = plsc.load_gather(u2pad, [vbase + dd])
                    acc1 = acc1 + v_d * u1_d
                    acc2 = acc2 + v_d * u2_d
                    ts = tuple(
                        ts[n] + v_d * plsc.load_gather(
                            negpad, [nbase + (dd + n * D)])
                        for n in range(NH))
                    return acc1, acc2, ts

                def dbody_b(dd, ts):
                    v_d = plsc.load_gather(vpad, [vbase + dd])
                    return tuple(
                        ts[n] + v_d * plsc.load_gather(
                            negpad, [nbase + (dd + (NH + n) * D)])
                        for n in range(NNEG - NH))

                s1, s2, ts_a = lax.fori_loop(
                    0, D, dbody_a, (zero, zero, (zero,) * NH))
                ts_b = lax.fori_loop(0, D, dbody_b, (zero,) * (NNEG - NH))
                ts = ts_a + ts_b

                smax = zero
                prod = jnp.full((L,), 1.0, jnp.float32)
                for t in ts:
                    smax = smax + jnp.maximum(t, 0.0)
                    prod = prod * (1.0 + jnp.exp(-jnp.abs(t)))
                f1 = 1.0 + jnp.exp(-jnp.abs(s1))
                f2 = 1.0 + jnp.exp(-jnp.abs(s2))
                wg = wgs[g]
                sc1 = (jnp.maximum(-s1, 0.0) + smax + _vlog(prod * f1)) * wg
                sc2 = (jnp.maximum(-s2, 0.0) + smax + _vlog(prod * f2)) * wg
                o1[pl.ds(g * L, L)] = sc1
                o2[pl.ds(g * L, L)] = sc2

            pltpu.sync_copy(o1, out1_hbm.at[pl.ds(b0, C)])
            pltpu.sync_copy(o2, out2_hbm.at[pl.ds(b0, C)])

        issue(0)

        def chunk_body(ci, carry):
            drain()
            relayout()
            wgs = [wbuf[pl.ds(g * L, L)] for g in range(GPC)]

            @pl.when(ci < NCHUNK - 1)
            def _issue_next():
                issue(ci + 1)

            compute_store(ci, wgs)
            return carry

        lax.fori_loop(0, NCHUNK, chunk_body, 0)

    return sc_kernel


_SC_KERNEL = _make_sc_kernel()


@jax.jit
def kernel(pos, neg, W_emb, W_ctx):
    pos_v = pos[:, 0].astype(jnp.int32)
    pos_u = pos[:, 1].astype(jnp.int32)
    w = pos[:, 2]
    neg2d = neg.astype(jnp.int32).reshape(NEG_ROWS_TOT, IDXW)
    return _SC_KERNEL(pos_v, pos_u, w, neg2d, W_emb, W_ctx)


# speedup vs baseline: 1.1217x; 1.1030x over previous
import functools

import jax
import jax.numpy as jnp
from jax import lax
from jax.experimental import pallas as pl
from jax.experimental.pallas import tpu as pltpu
from jax.experimental.pallas import tpu_sc as plsc

B = 16384
NNEG = 20
V = 1000000
D = 64
L = 16

NW = 32
BPW = B // NW
C = 32
NCHUNK = BPW // C
GPC = C // L

LN2 = 0.6931471805599453
SQRTH = 0.7071067811865476


def _vlog(x):
    bits = lax.bitcast_convert_type(x, jnp.int32)
    e = lax.shift_right_logical(bits, 23) - 127
    m = lax.bitcast_convert_type(
        (bits & 0x007FFFFF) | 0x3F800000, jnp.float32)
    ef = e.astype(jnp.float32)
    small = m < (2.0 * SQRTH)
    ef = jnp.where(small, ef, ef + 1.0)
    m = jnp.where(small, m, 0.5 * m)
    r = m - 1.0
    z = r * r
    p = 7.0376836292e-2
    p = p * r + -1.1514610310e-1
    p = p * r + 1.1676998740e-1
    p = p * r + -1.2420140846e-1
    p = p * r + 1.4249322787e-1
    p = p * r + -1.6668057665e-1
    p = p * r + 2.0000714765e-1
    p = p * r + -2.4999993993e-1
    p = p * r + 3.3333331174e-1
    y = r * z * p - 0.5 * z + r
    return y + ef * LN2


def _make_sc_kernel():
    mesh = plsc.VectorSubcoreMesh(core_axis_name="c", subcore_axis_name="s")

    @functools.partial(
        pl.kernel,
        out_type=(
            jax.ShapeDtypeStruct((B,), jnp.float32),
            jax.ShapeDtypeStruct((B,), jnp.float32),
        ),
        mesh=mesh,
        compiler_params=pltpu.CompilerParams(
            use_tc_tiling_on_sc=False, needs_layout_passes=False),
        scratch_types=[
            pltpu.VMEM((BPW,), jnp.int32),
            pltpu.VMEM((BPW,), jnp.int32),
            pltpu.VMEM((BPW,), jnp.float32),
            pltpu.VMEM((NNEG, BPW), jnp.int32),
            pltpu.VMEM((C, D), jnp.float32),
            pltpu.VMEM((C, D), jnp.float32),
            pltpu.VMEM((C, D), jnp.float32),
            pltpu.VMEM((NNEG, C, D), jnp.float32),
            pltpu.VMEM((C, D), jnp.float32),
            pltpu.VMEM((C, D), jnp.float32),
            pltpu.VMEM((C, D), jnp.float32),
            pltpu.VMEM((NNEG, C, D), jnp.float32),
            pltpu.VMEM((C,), jnp.float32),
            pltpu.VMEM((C,), jnp.float32),
            pltpu.SemaphoreType.DMA,
            pltpu.SemaphoreType.DMA,
            pltpu.SemaphoreType.DMA,
        ],
    )
    def sc_kernel(pos_v_hbm, pos_u_hbm, w_hbm, neg_hbm, emb_hbm, ctx_hbm,
                  out1_hbm, out2_hbm,
                  idxv, idxu, wbuf, negidx,
                  vr0, u1r0, u2r0, nr0, vr1, u1r1, u2r1, nr1,
                  o1, o2, sem0, sem1, semi):
        wid = lax.axis_index("s") * 2 + lax.axis_index("c")
        base = pl.multiple_of(wid * BPW, BPW)
        iota = lax.iota(jnp.int32, L)
        zero_i = jnp.zeros((L,), jnp.int32)

        bufs = ((vr0, u1r0, u2r0, nr0, sem0), (vr1, u1r1, u2r1, nr1, sem1))

        cps = [
            pltpu.async_copy(pos_v_hbm.at[pl.ds(base, BPW)], idxv, semi),
            pltpu.async_copy(pos_u_hbm.at[pl.ds(base, BPW)], idxu, semi),
            pltpu.async_copy(w_hbm.at[pl.ds(base, BPW)], wbuf, semi),
            pltpu.async_copy(neg_hbm.at[wid], negidx, semi),
        ]
        for cp in cps:
            cp.wait()

        def transfers(ci, s):
            vr, u1r, u2r, nr, sem = bufs[s]
            c0 = pl.multiple_of(ci * C, C)
            yield emb_hbm.at[idxv.at[pl.ds(c0, C)]], vr, sem
            yield emb_hbm.at[idxu.at[pl.ds(c0, C)]], u1r, sem
            yield ctx_hbm.at[idxu.at[pl.ds(c0, C)]], u2r, sem
            for n in range(NNEG):
                yield ctx_hbm.at[negidx.at[n, pl.ds(c0, C)]], nr.at[n], sem

        def issue(ci, s):
            for src, dst, sem in transfers(ci, s):
                pltpu.async_copy(src, dst, sem)

        def drain(ci, s):
            for src, dst, sem in transfers(ci, s):
                pltpu.make_async_copy(src, dst, sem).wait()

        def compute_store(ci, s):
            vr, u1r, u2r, nr, _ = bufs[s]
            c0 = pl.multiple_of(ci * C, C)
            zero = jnp.zeros((L,), jnp.float32)
            for g in range(GPC):
                bi = iota + (g * L)
                NH = NNEG // 2

                def dbody_a(dd, acc):
                    acc1, acc2, ts = acc
                    dv = (iota + dd) & (D - 1)
                    v_d = plsc.load_gather(vr, [bi, dv])
                    u1_d = plsc.load_gather(u1r, [bi, dv])
                    u2_d = plsc.load_gather(u2r, [bi, dv])
                    acc1 = acc1 + v_d * u1_d
                    acc2 = acc2 + v_d * u2_d
                    ts = tuple(
                        ts[n] + v_d * plsc.load_gather(
                            nr, [zero_i + n, bi, dv])
                        for n in range(NH))
                    return acc1, acc2, ts

                def dbody_b(dd, ts):
                    dv = (iota + dd) & (D - 1)
                    v_d = plsc.load_gather(vr, [bi, dv])
                    return tuple(
                        ts[n] + v_d * plsc.load_gather(
                            nr, [zero_i + (NH + n), bi, dv])
                        for n in range(NNEG - NH))

                s1, s2, ts_a = lax.fori_loop(
                    0, D, dbody_a, (zero, zero, (zero,) * NH))
                ts_b = lax.fori_loop(0, D, dbody_b, (zero,) * (NNEG - NH))
                ts = ts_a + ts_b

                smax = zero
                prod = jnp.full((L,), 1.0, jnp.float32)
                for t in ts:
                    smax = smax + jnp.maximum(t, 0.0)
                    prod = prod * (1.0 + jnp.exp(-jnp.abs(t)))
                f1 = 1.0 + jnp.exp(-jnp.abs(s1))
                f2 = 1.0 + jnp.exp(-jnp.abs(s2))
                wg = wbuf[pl.ds(c0 + g * L, L)]
                sc1 = (jnp.maximum(-s1, 0.0) + smax + _vlog(prod * f1)) * wg
                sc2 = (jnp.maximum(-s2, 0.0) + smax + _vlog(prod * f2)) * wg
                o1[pl.ds(g * L, L)] = sc1
                o2[pl.ds(g * L, L)] = sc2

            b0 = pl.multiple_of(base + ci * C, C)
            pltpu.sync_copy(o1, out1_hbm.at[pl.ds(b0, C)])
            pltpu.sync_copy(o2, out2_hbm.at[pl.ds(b0, C)])

        issue(0, 0)

        def chunk_pair(i, carry):
            ci0 = i * 2
            issue(ci0 + 1, 1)
            drain(ci0, 0)
            compute_store(ci0, 0)

            @pl.when(ci0 + 2 < NCHUNK)
            def _issue_next_even():
                issue(ci0 + 2, 0)

            drain(ci0 + 1, 1)
            compute_store(ci0 + 1, 1)
            return carry

        lax.fori_loop(0, NCHUNK // 2, chunk_pair, 0)

    return sc_kernel


_SC_KERNEL = _make_sc_kernel()


@jax.jit
def kernel(pos, neg, W_emb, W_ctx):
    pos_v = pos[:, 0].astype(jnp.int32)
    pos_u = pos[:, 1].astype(jnp.int32)
    w = pos[:, 2]
    neg3d = neg.astype(jnp.int32).reshape(NW, BPW, NNEG).transpose(0, 2, 1)
    return _SC_KERNEL(pos_v, pos_u, w, neg3d, W_emb, W_ctx)
